# baseline (speedup 1.0000x reference)
import numpy as np
import jax
import jax.numpy as jnp
from jax.experimental import pallas as pl
from jax.experimental.pallas import tpu as pltpu

B = 16384
D = 64
INDEG = 8
HID = 16
G = 8
NG = D // G
W = G * HID
BT = 2048

_ADJ = np.asarray(
    [sorted({j} | {(j + m) % D for m in range(1, 8)}) for j in range(D)],
    dtype=np.int32,
)

_TILE16 = np.zeros((HID, D * HID), np.float32)
for _d in range(D):
    _TILE16[:, _d * HID:(_d + 1) * HID] = np.eye(HID)
_EYEMASK = np.zeros((D, D * HID), np.float32)
for _d in range(D):
    _EYEMASK[_d, _d * HID:(_d + 1) * HID] = 1.0
_OHROW = np.zeros((D, D * INDEG), np.float32)
for _d in range(D):
    for _i in range(INDEG):
        _OHROW[_ADJ[_d, _i], _d * INDEG + _i] = 1.0
_BIGMASK = np.zeros((D * INDEG, D * HID), np.float32)
for _d in range(D):
    _BIGMASK[_d * INDEG:(_d + 1) * INDEG, _d * HID:(_d + 1) * HID] = 1.0
_TILE16G = np.zeros((HID, W), np.float32)
for _m in range(G):
    _TILE16G[:, _m * HID:(_m + 1) * HID] = np.eye(HID)
_EYE128 = np.zeros((W, W), np.float32)
for _n in range(G):
    _EYE128[_n * HID:(_n + 1) * HID, _n * HID:(_n + 1) * HID] = 1.0


def _leaky(x):
    return jnp.maximum(x, 0.01 * x)


def _mlp_kernel(x_ref, w0_ref, b0_ref, w1_ref, b1_ref, w2_ref,
                tile16_ref, eyemask_ref, ohrow_ref, bigmask_ref,
                tile16g_ref, eye128_ref,
                o_ref, w0s, b0s, w1s, b1s, w2s):
    @pl.when(pl.program_id(0) == 0)
    def _prep():
        tile16 = tile16_ref[...]
        eyemask = eyemask_ref[...]
        w0flat = w0_ref[...].reshape(D * INDEG, HID)
        w0tiled = jnp.dot(w0flat, tile16,
                          preferred_element_type=jnp.float32) * bigmask_ref[...]
        w0mat = jnp.dot(ohrow_ref[...], w0tiled,
                        preferred_element_type=jnp.float32)
        w0s[...] = w0mat.astype(jnp.bfloat16)
        b0m = jnp.dot(b0_ref[...], tile16,
                      preferred_element_type=jnp.float32) * eyemask
        b0s[...] = jnp.sum(b0m, axis=0, keepdims=True)
        b1m = jnp.dot(b1_ref[...], tile16,
                      preferred_element_type=jnp.float32) * eyemask
        b1s[...] = jnp.sum(b1m, axis=0, keepdims=True)
        w1flat = w1_ref[...].reshape(D * HID, HID)
        tile16g = tile16g_ref[...]
        eye128 = eye128_ref[...]
        for g in range(NG):
            blk = jnp.dot(w1flat[g * W:(g + 1) * W, :], tile16g,
                          preferred_element_type=jnp.float32) * eye128
            w1s[g] = blk.astype(jnp.bfloat16)
        w2d = jnp.dot(w2_ref[...].reshape(D, HID), tile16,
                      preferred_element_type=jnp.float32) * eyemask
        w2s[...] = w2d.T.astype(jnp.bfloat16)

    x = x_ref[...].astype(jnp.bfloat16)
    h1s = []
    for g in range(NG):
        h0 = jnp.dot(x, w0s[:, g * W:(g + 1) * W],
                     preferred_element_type=jnp.float32)
        h0 = _leaky(h0 + b0s[:, g * W:(g + 1) * W])
        h1 = jnp.dot(h0.astype(jnp.bfloat16), w1s[g],
                     preferred_element_type=jnp.float32)
        h1 = _leaky(h1 + b1s[:, g * W:(g + 1) * W])
        h1s.append(h1.astype(jnp.bfloat16))
    h1f = jnp.concatenate(h1s, axis=1)
    og = jnp.dot(h1f, w2s[...], preferred_element_type=jnp.float32)
    o_ref[...] = _leaky(og)


def kernel(state, W0, b0, W1, b1, W2):
    full = lambda shape: pl.BlockSpec(shape, lambda i: (0,) * len(shape))
    return pl.pallas_call(
        _mlp_kernel,
        grid=(B // BT,),
        in_specs=[
            pl.BlockSpec((BT, D), lambda i: (i, 0)),
            full((D, INDEG, HID)),
            full((D, HID)),
            full((D, HID, HID)),
            full((D, HID)),
            full((D, HID)),
            full((HID, D * HID)),
            full((D, D * HID)),
            full((D, D * INDEG)),
            full((D * INDEG, D * HID)),
            full((HID, W)),
            full((W, W)),
        ],
        out_specs=pl.BlockSpec((BT, D), lambda i: (i, 0)),
        out_shape=jax.ShapeDtypeStruct((B, D), state.dtype),
        scratch_shapes=[
            pltpu.VMEM((D, D * HID), jnp.bfloat16),
            pltpu.VMEM((1, D * HID), jnp.float32),
            pltpu.VMEM((NG, W, W), jnp.bfloat16),
            pltpu.VMEM((1, D * HID), jnp.float32),
            pltpu.VMEM((D * HID, D), jnp.bfloat16),
        ],
    )(state, W0, b0, W1, b1, W2[..., 0],
      jnp.asarray(_TILE16), jnp.asarray(_EYEMASK), jnp.asarray(_OHROW),
      jnp.asarray(_BIGMASK), jnp.asarray(_TILE16G), jnp.asarray(_EYE128))


# speedup vs baseline: 1.0081x; 1.0081x over previous
import numpy as np
import jax
import jax.numpy as jnp
from jax.experimental import pallas as pl

B = 16384
D = 64
INDEG = 8
HID = 16
G = 8
NG = D // G
W = G * HID
BT = 2048

_ADJ = np.asarray(
    [sorted({j} | {(j + m) % D for m in range(1, 8)}) for j in range(D)],
    dtype=np.int32,
)

_TILE16 = np.zeros((HID, D * HID), np.float32)
for _d in range(D):
    _TILE16[:, _d * HID:(_d + 1) * HID] = np.eye(HID)
_EYEMASK = np.zeros((D, D * HID), np.float32)
for _d in range(D):
    _EYEMASK[_d, _d * HID:(_d + 1) * HID] = 1.0
_OHROW = np.zeros((D, D * INDEG), np.float32)
for _d in range(D):
    for _i in range(INDEG):
        _OHROW[_ADJ[_d, _i], _d * INDEG + _i] = 1.0
_BIGMASK = np.zeros((D * INDEG, D * HID), np.float32)
for _d in range(D):
    _BIGMASK[_d * INDEG:(_d + 1) * INDEG, _d * HID:(_d + 1) * HID] = 1.0
_TILE16G = np.zeros((HID, W), np.float32)
for _m in range(G):
    _TILE16G[:, _m * HID:(_m + 1) * HID] = np.eye(HID)
_EYE128 = np.zeros((W, W), np.float32)
for _n in range(G):
    _EYE128[_n * HID:(_n + 1) * HID, _n * HID:(_n + 1) * HID] = 1.0


def _leaky(x):
    return jnp.maximum(x, 0.01 * x)


def _prep_kernel(w0_ref, b0_ref, w1_ref, b1_ref, w2_ref,
                 tile16_ref, eyemask_ref, ohrow_ref, bigmask_ref,
                 tile16g_ref, eye128_ref,
                 w0m_ref, b0m_ref, w1m_ref, b1m_ref, w2m_ref):
    tile16 = tile16_ref[...]
    eyemask = eyemask_ref[...]
    w0flat = w0_ref[...].reshape(D * INDEG, HID)
    w0tiled = jnp.dot(w0flat, tile16,
                      preferred_element_type=jnp.float32) * bigmask_ref[...]
    w0mat = jnp.dot(ohrow_ref[...], w0tiled,
                    preferred_element_type=jnp.float32)
    w0m_ref[...] = w0mat.astype(jnp.bfloat16)
    b0t = jnp.dot(b0_ref[...], tile16,
                  preferred_element_type=jnp.float32) * eyemask
    b0m_ref[...] = jnp.sum(b0t, axis=0, keepdims=True)
    b1t = jnp.dot(b1_ref[...], tile16,
                  preferred_element_type=jnp.float32) * eyemask
    b1m_ref[...] = jnp.sum(b1t, axis=0, keepdims=True)
    w1flat = w1_ref[...].reshape(D * HID, HID)
    tile16g = tile16g_ref[...]
    eye128 = eye128_ref[...]
    for g in range(NG):
        blk = jnp.dot(w1flat[g * W:(g + 1) * W, :], tile16g,
                      preferred_element_type=jnp.float32) * eye128
        w1m_ref[g] = blk.astype(jnp.bfloat16)
    w2d = jnp.dot(w2_ref[...].reshape(D, HID), tile16,
                  preferred_element_type=jnp.float32) * eyemask
    w2m_ref[...] = w2d.T.astype(jnp.bfloat16)


def _mlp_kernel(x_ref, w0_ref, b0_ref, w1_ref, b1_ref, w2_ref, o_ref):
    x = x_ref[...].astype(jnp.bfloat16)
    h1s = []
    for g in range(NG):
        h0 = jnp.dot(x, w0_ref[:, g * W:(g + 1) * W],
                     preferred_element_type=jnp.float32)
        h0 = _leaky(h0 + b0_ref[:, g * W:(g + 1) * W])
        h1 = jnp.dot(h0.astype(jnp.bfloat16), w1_ref[g],
                     preferred_element_type=jnp.float32)
        h1 = _leaky(h1 + b1_ref[:, g * W:(g + 1) * W])
        h1s.append(h1.astype(jnp.bfloat16))
    h1f = jnp.concatenate(h1s, axis=1)
    og = jnp.dot(h1f, w2_ref[...], preferred_element_type=jnp.float32)
    o_ref[...] = _leaky(og)


def kernel(state, W0, b0, W1, b1, W2):
    full = lambda shape: pl.BlockSpec(shape, lambda *_: (0,) * len(shape))
    w0m, b0m, w1m, b1m, w2m = pl.pallas_call(
        _prep_kernel,
        in_specs=[
            full((D, INDEG, HID)),
            full((D, HID)),
            full((D, HID, HID)),
            full((D, HID)),
            full((D, HID)),
            full((HID, D * HID)),
            full((D, D * HID)),
            full((D, D * INDEG)),
            full((D * INDEG, D * HID)),
            full((HID, W)),
            full((W, W)),
        ],
        out_specs=[
            full((D, D * HID)),
            full((1, D * HID)),
            full((NG, W, W)),
            full((1, D * HID)),
            full((D * HID, D)),
        ],
        out_shape=[
            jax.ShapeDtypeStruct((D, D * HID), jnp.bfloat16),
            jax.ShapeDtypeStruct((1, D * HID), jnp.float32),
            jax.ShapeDtypeStruct((NG, W, W), jnp.bfloat16),
            jax.ShapeDtypeStruct((1, D * HID), jnp.float32),
            jax.ShapeDtypeStruct((D * HID, D), jnp.bfloat16),
        ],
    )(W0, b0, W1, b1, W2[..., 0],
      jnp.asarray(_TILE16), jnp.asarray(_EYEMASK), jnp.asarray(_OHROW),
      jnp.asarray(_BIGMASK), jnp.asarray(_TILE16G), jnp.asarray(_EYE128))

    return pl.pallas_call(
        _mlp_kernel,
        grid=(B // BT,),
        in_specs=[
            pl.BlockSpec((BT, D), lambda i: (i, 0)),
            full((D, D * HID)),
            full((1, D * HID)),
            full((NG, W, W)),
            full((1, D * HID)),
            full((D * HID, D)),
        ],
        out_specs=pl.BlockSpec((BT, D), lambda i: (i, 0)),
        out_shape=jax.ShapeDtypeStruct((B, D), state.dtype),
    )(state, w0m, b0m, w1m, b1m, w2m)


# speedup vs baseline: 1.0181x; 1.0100x over previous
import numpy as np
import jax
import jax.numpy as jnp
from jax.experimental import pallas as pl

B = 16384
D = 64
INDEG = 8
HID = 16
G = 8
NG = D // G
W = G * HID
BT = 4096

_ADJ = np.asarray(
    [sorted({j} | {(j + m) % D for m in range(1, 8)}) for j in range(D)],
    dtype=np.int32,
)

_TILE16 = np.zeros((HID, D * HID), np.float32)
for _d in range(D):
    _TILE16[:, _d * HID:(_d + 1) * HID] = np.eye(HID)
_EYEMASK = np.zeros((D, D * HID), np.float32)
for _d in range(D):
    _EYEMASK[_d, _d * HID:(_d + 1) * HID] = 1.0
_OHROW = np.zeros((D, D * INDEG), np.float32)
for _d in range(D):
    for _i in range(INDEG):
        _OHROW[_ADJ[_d, _i], _d * INDEG + _i] = 1.0
_BIGMASK = np.zeros((D * INDEG, D * HID), np.float32)
for _d in range(D):
    _BIGMASK[_d * INDEG:(_d + 1) * INDEG, _d * HID:(_d + 1) * HID] = 1.0
_TILE16G = np.zeros((HID, W), np.float32)
for _m in range(G):
    _TILE16G[:, _m * HID:(_m + 1) * HID] = np.eye(HID)
_EYE128 = np.zeros((W, W), np.float32)
for _n in range(G):
    _EYE128[_n * HID:(_n + 1) * HID, _n * HID:(_n + 1) * HID] = 1.0


def _leaky(x):
    return jnp.maximum(x, 0.01 * x)


def _prep_kernel(w0_ref, b0_ref, w1_ref, b1_ref, w2_ref,
                 tile16_ref, eyemask_ref, ohrow_ref, bigmask_ref,
                 tile16g_ref, eye128_ref,
                 w0m_ref, b0m_ref, w1m_ref, b1m_ref, w2m_ref):
    tile16 = tile16_ref[...]
    eyemask = eyemask_ref[...]
    w0flat = w0_ref[...].reshape(D * INDEG, HID)
    w0tiled = jnp.dot(w0flat, tile16,
                      preferred_element_type=jnp.float32) * bigmask_ref[...]
    w0mat = jnp.dot(ohrow_ref[...], w0tiled,
                    preferred_element_type=jnp.float32)
    w0m_ref[...] = w0mat.astype(jnp.bfloat16)
    b0t = jnp.dot(b0_ref[...], tile16,
                  preferred_element_type=jnp.float32) * eyemask
    b0m_ref[...] = jnp.sum(b0t, axis=0, keepdims=True)
    b1t = jnp.dot(b1_ref[...], tile16,
                  preferred_element_type=jnp.float32) * eyemask
    b1m_ref[...] = jnp.sum(b1t, axis=0, keepdims=True)
    w1flat = w1_ref[...].reshape(D * HID, HID)
    tile16g = tile16g_ref[...]
    eye128 = eye128_ref[...]
    for g in range(NG):
        blk = jnp.dot(w1flat[g * W:(g + 1) * W, :], tile16g,
                      preferred_element_type=jnp.float32) * eye128
        w1m_ref[g] = blk.astype(jnp.bfloat16)
    w2d = jnp.dot(w2_ref[...].reshape(D, HID), tile16,
                  preferred_element_type=jnp.float32) * eyemask
    w2m_ref[...] = w2d.T.astype(jnp.bfloat16)


def _mlp_kernel(x_ref, w0_ref, b0_ref, w1_ref, b1_ref, w2_ref, o_ref):
    x = x_ref[...].astype(jnp.bfloat16)
    h1s = []
    for g in range(NG):
        h0 = jnp.dot(x, w0_ref[:, g * W:(g + 1) * W],
                     preferred_element_type=jnp.float32)
        h0 = _leaky(h0 + b0_ref[:, g * W:(g + 1) * W])
        h1 = jnp.dot(h0.astype(jnp.bfloat16), w1_ref[g],
                     preferred_element_type=jnp.float32)
        h1 = _leaky(h1 + b1_ref[:, g * W:(g + 1) * W])
        h1s.append(h1.astype(jnp.bfloat16))
    h1f = jnp.concatenate(h1s, axis=1)
    og = jnp.dot(h1f, w2_ref[...], preferred_element_type=jnp.float32)
    o_ref[...] = _leaky(og)


def kernel(state, W0, b0, W1, b1, W2):
    full = lambda shape: pl.BlockSpec(shape, lambda *_: (0,) * len(shape))
    w0m, b0m, w1m, b1m, w2m = pl.pallas_call(
        _prep_kernel,
        in_specs=[
            full((D, INDEG, HID)),
            full((D, HID)),
            full((D, HID, HID)),
            full((D, HID)),
            full((D, HID)),
            full((HID, D * HID)),
            full((D, D * HID)),
            full((D, D * INDEG)),
            full((D * INDEG, D * HID)),
            full((HID, W)),
            full((W, W)),
        ],
        out_specs=[
            full((D, D * HID)),
            full((1, D * HID)),
            full((NG, W, W)),
            full((1, D * HID)),
            full((D * HID, D)),
        ],
        out_shape=[
            jax.ShapeDtypeStruct((D, D * HID), jnp.bfloat16),
            jax.ShapeDtypeStruct((1, D * HID), jnp.float32),
            jax.ShapeDtypeStruct((NG, W, W), jnp.bfloat16),
            jax.ShapeDtypeStruct((1, D * HID), jnp.float32),
            jax.ShapeDtypeStruct((D * HID, D), jnp.bfloat16),
        ],
    )(W0, b0, W1, b1, W2[..., 0],
      jnp.asarray(_TILE16), jnp.asarray(_EYEMASK), jnp.asarray(_OHROW),
      jnp.asarray(_BIGMASK), jnp.asarray(_TILE16G), jnp.asarray(_EYE128))

    return pl.pallas_call(
        _mlp_kernel,
        grid=(B // BT,),
        in_specs=[
            pl.BlockSpec((BT, D), lambda i: (i, 0)),
            full((D, D * HID)),
            full((1, D * HID)),
            full((NG, W, W)),
            full((1, D * HID)),
            full((D * HID, D)),
        ],
        out_specs=pl.BlockSpec((BT, D), lambda i: (i, 0)),
        out_shape=jax.ShapeDtypeStruct((B, D), state.dtype),
    )(state, w0m, b0m, w1m, b1m, w2m)
